# baseline (speedup 1.0000x reference)
import functools

import jax
import jax.numpy as jnp
import numpy as np
from jax import lax
from jax.experimental import pallas as pl
from jax.experimental.pallas import tpu as pltpu
from jax.experimental.pallas import tpu_sc as plsc

TOP_K = 30
NUM_RBF = 16
MAX_REL = 32

_PAIRS = [(1, 1),
          (0, 0), (2, 2), (4, 4), (3, 3), (1, 0), (1, 2), (1, 4), (1, 3),
          (0, 2), (0, 4), (0, 3), (3, 2), (3, 4), (4, 2), (0, 1), (2, 1),
          (4, 1), (3, 1), (2, 0), (4, 0), (3, 0), (2, 3), (4, 3), (2, 4)]
_NP = len(_PAIRS)


def _static_mats():
    p1 = np.zeros((16, 3 * _NP), np.float32)
    p2 = np.zeros((16, 3 * _NP), np.float32)
    smat = np.zeros((3 * _NP, _NP), np.float32)
    for p, (a, b) in enumerate(_PAIRS):
        for c in range(3):
            p1[3 * a + c, 3 * p + c] = 1.0
            p2[3 * b + c, 3 * p + c] = 1.0
            smat[3 * p + c, p] = 1.0
    mu16 = 2.0 + np.arange(NUM_RBF, dtype=np.float32) * (20.0 / (NUM_RBF - 1))
    muw = np.repeat(mu16, _NP)[None, :]
    return (jnp.asarray(p1), jnp.asarray(p2), jnp.asarray(smat),
            jnp.asarray(muw))


def _topk_body(x16_ref, xt_ref, eidx_ref):
    x16 = x16_ref[0]
    xt = xt_ref[0]
    d2 = None
    for c in range(3):
        xb = x16[:, 3 + c:4 + c]
        xa = xt[3 + c:4 + c, :]
        df = xb - xa
        t = df * df
        d2 = t if d2 is None else d2 + t
    dcur = jnp.sqrt(d2 + 1e-6)
    iota = lax.broadcasted_iota(jnp.int32, dcur.shape, 1)
    for k in range(TOP_K):
        m = jnp.min(dcur, axis=1, keepdims=True)
        cand = jnp.where(dcur == m, iota, jnp.int32(1 << 30))
        idx = jnp.min(cand, axis=1, keepdims=True)
        eidx_ref[0, :, k:k + 1] = idx
        if k < TOP_K - 1:
            dcur = jnp.where(iota == idx, jnp.float32(1e30), dcur)


def _edge_body(own_ref, nb_ref, ei_ref, p1_ref, p2_ref, s_ref,
               mu_ref, ew_ref, pw_ref, g_ref, b_ref, pb_ref, e_ref):
    ownb = own_ref[0]
    nb = nb_ref[0]
    kk = nb.shape[0] // ownb.shape[0]
    own = jnp.broadcast_to(ownb[:, None, :],
                           (ownb.shape[0], kk, 16)).reshape(nb.shape)
    own_e = jnp.dot(own, p1_ref[...], preferred_element_type=jnp.float32, precision=lax.Precision.HIGHEST)
    nb_e = jnp.dot(nb, p2_ref[...], preferred_element_type=jnp.float32, precision=lax.Precision.HIGHEST)
    diff = own_e - nb_e
    d2 = jnp.dot(diff * diff, s_ref[...], preferred_element_type=jnp.float32, precision=lax.Precision.HIGHEST)
    d = jnp.sqrt(d2 + 1e-6)
    dw = jnp.concatenate([d] * NUM_RBF, axis=1)
    z = (dw - mu_ref[...]) * jnp.float32(NUM_RBF / 20.0)
    rbf = jnp.exp(-(z * z))
    y = jnp.dot(rbf, ew_ref[...], preferred_element_type=jnp.float32)
    m = jnp.mean(y, axis=1, keepdims=True)
    v = jnp.mean((y - m) ** 2, axis=1, keepdims=True)
    yn = (y - m) * lax.rsqrt(v + 1e-5) * g_ref[...] + b_ref[...]
    rbl = ownb.shape[0]
    rows = (pl.program_id(1) * rbl
            + lax.broadcasted_iota(jnp.int32, (rbl, kk), 0))
    dcl = jnp.clip(rows - ei_ref[0] + MAX_REL, 0, 2 * MAX_REL)
    oh = (dcl[:, :, None]
          == lax.broadcasted_iota(jnp.int32, (1, 1, 2 * MAX_REL + 1), 2)
          ).astype(jnp.float32).reshape(nb.shape[0], 2 * MAX_REL + 1)
    pos = jnp.dot(oh, pw_ref[...], preferred_element_type=jnp.float32)
    out = yn + pos + pb_ref[...]
    e_ref[0] = out.reshape(e_ref.shape[1:])


def _node_body(x16_ref, w_ref, g_ref, b_ref, v_ref):
    x = x16_ref[0]

    def col(i):
        return x[:, i:i + 1]

    zero = jnp.zeros((1, 1), jnp.float32)

    def shift_up(v):
        return jnp.concatenate([v[1:, :], zero], axis=0)

    def shift_down(v):
        return jnp.concatenate([zero, v[:-1, :]], axis=0)

    def norm3(v):
        n = jnp.sqrt(v[0] * v[0] + v[1] * v[1] + v[2] * v[2])
        inv = 1.0 / jnp.maximum(n, 1e-12)
        return [v[0] * inv, v[1] * inv, v[2] * inv]

    def cross(u, v):
        return [u[1] * v[2] - u[2] * v[1],
                u[2] * v[0] - u[0] * v[2],
                u[0] * v[1] - u[1] * v[0]]

    def dot3(u, v):
        return u[0] * v[0] + u[1] * v[1] + u[2] * v[2]

    n_at = [col(0), col(1), col(2)]
    ca_at = [col(3), col(4), col(5)]
    c_at = [col(6), col(7), col(8)]
    avec = norm3([ca_at[c] - n_at[c] for c in range(3)])
    bvec = norm3([c_at[c] - ca_at[c] for c in range(3)])
    cvec = norm3([shift_up(n_at[c]) - c_at[c] for c in range(3)])

    def dihed(u2, u1, u0):
        n2 = norm3(cross(u2, u1))
        n1 = norm3(cross(u1, u0))
        cosd = jnp.clip(dot3(n2, n1), -1.0 + 1e-7, 1.0 - 1e-7)
        sind = jnp.sign(dot3(u2, n1)) * jnp.sqrt(1.0 - cosd * cosd)
        return cosd, sind

    c_prev = [shift_down(cvec[i]) for i in range(3)]
    a_next = [shift_up(avec[i]) for i in range(3)]
    cph, sph = dihed(c_prev, avec, bvec)
    cps, sps = dihed(avec, bvec, cvec)
    com, som = dihed(bvec, cvec, a_next)
    r = lax.broadcasted_iota(jnp.int32, cph.shape, 0)
    first = r == 0
    last = r == (cph.shape[0] - 1)
    cph = jnp.where(first, 1.0, cph)
    sph = jnp.where(first, 0.0, sph)
    cps = jnp.where(last, 1.0, cps)
    sps = jnp.where(last, 0.0, sps)
    com = jnp.where(last, 1.0, com)
    som = jnp.where(last, 0.0, som)
    feat = jnp.concatenate([cph, cps, com, sph, sps, som], axis=1)
    y = jnp.dot(feat, w_ref[...], preferred_element_type=jnp.float32, precision=lax.Precision.HIGHEST)
    m = jnp.mean(y, axis=1, keepdims=True)
    v = jnp.mean((y - m) ** 2, axis=1, keepdims=True)
    v_ref[0] = (y - m) / jnp.sqrt(v + 1e-5) * g_ref[...] + b_ref[...]


def _sc_gather(table, gidx3, batch):
    info = plsc.get_sparse_core_info()
    nc, ns = info.num_cores, info.num_subcores
    nw = nc * ns
    chunks = gidx3.shape[1]
    per_w = chunks * 128
    rows = nw * per_w
    wpb = nw // batch
    mesh = plsc.VectorSubcoreMesh(core_axis_name="c", subcore_axis_name="s")

    @functools.partial(
        pl.kernel, mesh=mesh,
        compiler_params=pltpu.CompilerParams(use_tc_tiling_on_sc=False),
        out_type=jax.ShapeDtypeStruct((batch, rows // batch, 16), jnp.float32),
        scratch_types=[
            pltpu.VMEM((chunks, 128), jnp.int32),
            pltpu.VMEM((per_w, 16), jnp.float32),
            pltpu.SemaphoreType.DMA,
            pltpu.SemaphoreType.DMA,
        ])
    def gather_k(table_hbm, idx_hbm, out_hbm, idx_v, rows_v, sem0, sem1):
        wid = lax.axis_index("s") * nc + lax.axis_index("c")
        pltpu.sync_copy(idx_hbm.at[wid], idx_v)

        def body(g, carry):
            c0 = 2 * g
            c1 = 2 * g + 1
            cp0 = pltpu.make_async_copy(
                table_hbm.at[idx_v.at[c0]],
                rows_v.at[pl.ds(c0 * 128, 128)], sem0)
            cp1 = pltpu.make_async_copy(
                table_hbm.at[idx_v.at[c1]],
                rows_v.at[pl.ds(c1 * 128, 128)], sem1)
            cp0.start()
            cp1.start()
            cp0.wait()
            cp1.wait()
            return carry

        lax.fori_loop(0, chunks // 2, body, 0)
        pltpu.sync_copy(
            rows_v, out_hbm.at[wid // wpb, pl.ds((wid % wpb) * per_w, per_w)])

    return gather_k(table, gidx3)


def kernel(X, mask, L, pos_W, pos_b, edge_W, ne_g, ne_b, node_W, nn_g, nn_b):
    B, Ln = X.shape[0], X.shape[1]
    K = TOP_K
    F = edge_W.shape[1]
    x16 = jnp.concatenate(
        [X.reshape(B, Ln, 15), jnp.zeros((B, Ln, 1), jnp.float32)], axis=-1)
    xt = jnp.transpose(x16, (0, 2, 1))

    NF = node_W.shape[1]
    V = pl.pallas_call(
        _node_body,
        grid=(B,),
        in_specs=[
            pl.BlockSpec((1, Ln, 16), lambda b: (b, 0, 0)),
            pl.BlockSpec((6, NF), lambda b: (0, 0)),
            pl.BlockSpec((1, NF), lambda b: (0, 0)),
            pl.BlockSpec((1, NF), lambda b: (0, 0)),
        ],
        out_specs=pl.BlockSpec((1, Ln, NF), lambda b: (b, 0, 0)),
        out_shape=jax.ShapeDtypeStruct((B, Ln, NF), jnp.float32),
    )(x16, node_W, nn_g[None, :], nn_b[None, :])

    BL = 128
    eidx = pl.pallas_call(
        _topk_body,
        grid=(B, Ln // BL),
        in_specs=[
            pl.BlockSpec((1, BL, 16), lambda b, i: (b, i, 0)),
            pl.BlockSpec((1, 16, Ln), lambda b, i: (b, 0, 0)),
        ],
        out_specs=pl.BlockSpec((1, BL, K), lambda b, i: (b, i, 0)),
        out_shape=jax.ShapeDtypeStruct((B, Ln, K), jnp.int32),
    )(x16, xt)

    info = plsc.get_sparse_core_info()
    nw = info.num_cores * info.num_subcores
    gidx = (eidx + (jnp.arange(B, dtype=jnp.int32) * Ln)[:, None, None])
    gidx3 = gidx.reshape(nw, (B * Ln * K) // (nw * 128), 128)
    table = x16.reshape(B * Ln, 16)
    nb_flat = _sc_gather(table, gidx3, B)

    RB = 3840
    RBL = RB // K
    p1, p2, smat, muw = _static_mats()
    nrbf = NUM_RBF * _NP
    ew_mu = edge_W.reshape(_NP, NUM_RBF, F).transpose(1, 0, 2).reshape(nrbf, F)
    e_flat = pl.pallas_call(
        _edge_body,
        grid=(B, (Ln * K) // RB),
        in_specs=[
            pl.BlockSpec((1, RBL, 16), lambda b, i: (b, i, 0)),
            pl.BlockSpec((1, RB, 16), lambda b, i: (b, i, 0)),
            pl.BlockSpec((1, RBL, K), lambda b, i: (b, i, 0)),
            pl.BlockSpec(p1.shape, lambda b, i: (0, 0)),
            pl.BlockSpec(p2.shape, lambda b, i: (0, 0)),
            pl.BlockSpec(smat.shape, lambda b, i: (0, 0)),
            pl.BlockSpec((1, nrbf), lambda b, i: (0, 0)),
            pl.BlockSpec((nrbf, F), lambda b, i: (0, 0)),
            pl.BlockSpec((2 * MAX_REL + 1, F), lambda b, i: (0, 0)),
            pl.BlockSpec((1, F), lambda b, i: (0, 0)),
            pl.BlockSpec((1, F), lambda b, i: (0, 0)),
            pl.BlockSpec((1, F), lambda b, i: (0, 0)),
        ],
        out_specs=pl.BlockSpec((1, RBL, K, F), lambda b, i: (b, i, 0, 0)),
        out_shape=jax.ShapeDtypeStruct((B, Ln, K, F), jnp.float32),
    )(x16, nb_flat, eidx, p1, p2, smat, muw, ew_mu, pos_W,
      ne_g[None, :], ne_b[None, :], pos_b[None, :])

    return V, e_flat, eidx


# speedup vs baseline: 1.0297x; 1.0297x over previous
import functools

import jax
import jax.numpy as jnp
import numpy as np
from jax import lax
from jax.experimental import pallas as pl
from jax.experimental.pallas import tpu as pltpu
from jax.experimental.pallas import tpu_sc as plsc

TOP_K = 30
NUM_RBF = 16
MAX_REL = 32

_PAIRS = [(1, 1),
          (0, 0), (2, 2), (4, 4), (3, 3), (1, 0), (1, 2), (1, 4), (1, 3),
          (0, 2), (0, 4), (0, 3), (3, 2), (3, 4), (4, 2), (0, 1), (2, 1),
          (4, 1), (3, 1), (2, 0), (4, 0), (3, 0), (2, 3), (4, 3), (2, 4)]
_NP = len(_PAIRS)


def _static_mats():
    p1 = np.zeros((16, 3 * _NP), np.float32)
    p2 = np.zeros((16, 3 * _NP), np.float32)
    smat = np.zeros((3 * _NP, _NP), np.float32)
    for p, (a, b) in enumerate(_PAIRS):
        for c in range(3):
            p1[3 * a + c, 3 * p + c] = 1.0
            p2[3 * b + c, 3 * p + c] = 1.0
            smat[3 * p + c, p] = 1.0
    mu16 = 2.0 + np.arange(NUM_RBF, dtype=np.float32) * (20.0 / (NUM_RBF - 1))
    is2 = (NUM_RBF / 20.0) ** 2
    cmat = np.zeros((2 * _NP + 1, NUM_RBF * _NP), np.float32)
    for p in range(_NP):
        for j in range(NUM_RBF):
            col = NUM_RBF * p + j
            cmat[p, col] = -is2
            cmat[_NP + p, col] = 2.0 * mu16[j] * is2
            cmat[2 * _NP, col] = -mu16[j] * mu16[j] * is2
    return (jnp.asarray(p1), jnp.asarray(p2), jnp.asarray(smat),
            jnp.asarray(cmat))


def _topk_body(x16_ref, xt_ref, eidx_ref):
    x16 = x16_ref[0]
    xt = xt_ref[0]
    d2 = None
    for c in range(3):
        xb = x16[:, 3 + c:4 + c]
        xa = xt[3 + c:4 + c, :]
        df = xb - xa
        t = df * df
        d2 = t if d2 is None else d2 + t
    dcur = jnp.sqrt(d2 + 1e-6)
    iota = lax.broadcasted_iota(jnp.int32, dcur.shape, 1)
    for k in range(TOP_K):
        m = jnp.min(dcur, axis=1, keepdims=True)
        cand = jnp.where(dcur == m, iota, jnp.int32(1 << 30))
        idx = jnp.min(cand, axis=1, keepdims=True)
        eidx_ref[0, :, k:k + 1] = idx
        if k < TOP_K - 1:
            dcur = jnp.where(iota == idx, jnp.float32(1e30), dcur)


def _edge_body(own_ref, nb_ref, ei_ref, p1_ref, p2_ref, s_ref,
               c_ref, ew_ref, pw_ref, g_ref, b_ref, pb_ref, e_ref):
    ownb = own_ref[0]
    nb = nb_ref[0]
    kk = nb.shape[0] // ownb.shape[0]
    own = jnp.broadcast_to(ownb[:, None, :],
                           (ownb.shape[0], kk, 16)).reshape(nb.shape)
    own_e = jnp.dot(own, p1_ref[...], preferred_element_type=jnp.float32, precision=lax.Precision.HIGHEST)
    nb_e = jnp.dot(nb, p2_ref[...], preferred_element_type=jnp.float32, precision=lax.Precision.HIGHEST)
    diff = own_e - nb_e
    d2 = jnp.dot(diff * diff, s_ref[...], preferred_element_type=jnp.float32, precision=lax.Precision.HIGHEST)
    d2p = d2 + 1e-6
    d = jnp.sqrt(d2p)
    ones = jnp.full((nb.shape[0], 1), 1.0, jnp.float32)
    g = jnp.concatenate([d2p, d, ones], axis=1)
    zneg = jnp.dot(g, c_ref[...], preferred_element_type=jnp.float32, precision=lax.Precision.HIGHEST)
    rbf = jnp.exp(zneg)
    y = jnp.dot(rbf, ew_ref[...], preferred_element_type=jnp.float32)
    m = jnp.mean(y, axis=1, keepdims=True)
    v = jnp.mean((y - m) ** 2, axis=1, keepdims=True)
    yn = (y - m) * lax.rsqrt(v + 1e-5) * g_ref[...] + b_ref[...]
    rbl = ownb.shape[0]
    rows = (pl.program_id(1) * rbl
            + lax.broadcasted_iota(jnp.int32, (rbl, kk), 0))
    dcl = jnp.clip(rows - ei_ref[0] + MAX_REL, 0, 2 * MAX_REL)
    oh = (dcl[:, :, None]
          == lax.broadcasted_iota(jnp.int32, (1, 1, 2 * MAX_REL + 1), 2)
          ).astype(jnp.float32).reshape(nb.shape[0], 2 * MAX_REL + 1)
    pos = jnp.dot(oh, pw_ref[...], preferred_element_type=jnp.float32)
    out = yn + pos + pb_ref[...]
    e_ref[0] = out.reshape(e_ref.shape[1:])


def _node_body(x16_ref, w_ref, g_ref, b_ref, v_ref):
    x = x16_ref[0]

    def col(i):
        return x[:, i:i + 1]

    zero = jnp.zeros((1, 1), jnp.float32)

    def shift_up(v):
        return jnp.concatenate([v[1:, :], zero], axis=0)

    def shift_down(v):
        return jnp.concatenate([zero, v[:-1, :]], axis=0)

    def norm3(v):
        n = jnp.sqrt(v[0] * v[0] + v[1] * v[1] + v[2] * v[2])
        inv = 1.0 / jnp.maximum(n, 1e-12)
        return [v[0] * inv, v[1] * inv, v[2] * inv]

    def cross(u, v):
        return [u[1] * v[2] - u[2] * v[1],
                u[2] * v[0] - u[0] * v[2],
                u[0] * v[1] - u[1] * v[0]]

    def dot3(u, v):
        return u[0] * v[0] + u[1] * v[1] + u[2] * v[2]

    n_at = [col(0), col(1), col(2)]
    ca_at = [col(3), col(4), col(5)]
    c_at = [col(6), col(7), col(8)]
    avec = norm3([ca_at[c] - n_at[c] for c in range(3)])
    bvec = norm3([c_at[c] - ca_at[c] for c in range(3)])
    cvec = norm3([shift_up(n_at[c]) - c_at[c] for c in range(3)])

    def dihed(u2, u1, u0):
        n2 = norm3(cross(u2, u1))
        n1 = norm3(cross(u1, u0))
        cosd = jnp.clip(dot3(n2, n1), -1.0 + 1e-7, 1.0 - 1e-7)
        sind = jnp.sign(dot3(u2, n1)) * jnp.sqrt(1.0 - cosd * cosd)
        return cosd, sind

    c_prev = [shift_down(cvec[i]) for i in range(3)]
    a_next = [shift_up(avec[i]) for i in range(3)]
    cph, sph = dihed(c_prev, avec, bvec)
    cps, sps = dihed(avec, bvec, cvec)
    com, som = dihed(bvec, cvec, a_next)
    r = lax.broadcasted_iota(jnp.int32, cph.shape, 0)
    first = r == 0
    last = r == (cph.shape[0] - 1)
    cph = jnp.where(first, 1.0, cph)
    sph = jnp.where(first, 0.0, sph)
    cps = jnp.where(last, 1.0, cps)
    sps = jnp.where(last, 0.0, sps)
    com = jnp.where(last, 1.0, com)
    som = jnp.where(last, 0.0, som)
    feat = jnp.concatenate([cph, cps, com, sph, sps, som], axis=1)
    y = jnp.dot(feat, w_ref[...], preferred_element_type=jnp.float32, precision=lax.Precision.HIGHEST)
    m = jnp.mean(y, axis=1, keepdims=True)
    v = jnp.mean((y - m) ** 2, axis=1, keepdims=True)
    v_ref[0] = (y - m) / jnp.sqrt(v + 1e-5) * g_ref[...] + b_ref[...]


def _sc_gather(table, gidx3, batch):
    info = plsc.get_sparse_core_info()
    nc, ns = info.num_cores, info.num_subcores
    nw = nc * ns
    chunks = gidx3.shape[1]
    per_w = chunks * 128
    rows = nw * per_w
    wpb = nw // batch
    mesh = plsc.VectorSubcoreMesh(core_axis_name="c", subcore_axis_name="s")

    @functools.partial(
        pl.kernel, mesh=mesh,
        compiler_params=pltpu.CompilerParams(use_tc_tiling_on_sc=False),
        out_type=jax.ShapeDtypeStruct((batch, rows // batch, 16), jnp.float32),
        scratch_types=[
            pltpu.VMEM((chunks, 128), jnp.int32),
            pltpu.VMEM((per_w, 16), jnp.float32),
            pltpu.SemaphoreType.DMA,
            pltpu.SemaphoreType.DMA,
        ])
    def gather_k(table_hbm, idx_hbm, out_hbm, idx_v, rows_v, sem0, sem1):
        wid = lax.axis_index("s") * nc + lax.axis_index("c")
        pltpu.sync_copy(idx_hbm.at[wid], idx_v)

        def body(g, carry):
            c0 = 2 * g
            c1 = 2 * g + 1
            cp0 = pltpu.make_async_copy(
                table_hbm.at[idx_v.at[c0]],
                rows_v.at[pl.ds(c0 * 128, 128)], sem0)
            cp1 = pltpu.make_async_copy(
                table_hbm.at[idx_v.at[c1]],
                rows_v.at[pl.ds(c1 * 128, 128)], sem1)
            cp0.start()
            cp1.start()
            cp0.wait()
            cp1.wait()
            return carry

        lax.fori_loop(0, chunks // 2, body, 0)
        pltpu.sync_copy(
            rows_v, out_hbm.at[wid // wpb, pl.ds((wid % wpb) * per_w, per_w)])

    return gather_k(table, gidx3)


def kernel(X, mask, L, pos_W, pos_b, edge_W, ne_g, ne_b, node_W, nn_g, nn_b):
    B, Ln = X.shape[0], X.shape[1]
    K = TOP_K
    F = edge_W.shape[1]
    x16 = jnp.concatenate(
        [X.reshape(B, Ln, 15), jnp.zeros((B, Ln, 1), jnp.float32)], axis=-1)
    xt = jnp.transpose(x16, (0, 2, 1))

    NF = node_W.shape[1]
    V = pl.pallas_call(
        _node_body,
        grid=(B,),
        in_specs=[
            pl.BlockSpec((1, Ln, 16), lambda b: (b, 0, 0)),
            pl.BlockSpec((6, NF), lambda b: (0, 0)),
            pl.BlockSpec((1, NF), lambda b: (0, 0)),
            pl.BlockSpec((1, NF), lambda b: (0, 0)),
        ],
        out_specs=pl.BlockSpec((1, Ln, NF), lambda b: (b, 0, 0)),
        out_shape=jax.ShapeDtypeStruct((B, Ln, NF), jnp.float32),
    )(x16, node_W, nn_g[None, :], nn_b[None, :])

    BL = 128
    eidx = pl.pallas_call(
        _topk_body,
        grid=(B, Ln // BL),
        in_specs=[
            pl.BlockSpec((1, BL, 16), lambda b, i: (b, i, 0)),
            pl.BlockSpec((1, 16, Ln), lambda b, i: (b, 0, 0)),
        ],
        out_specs=pl.BlockSpec((1, BL, K), lambda b, i: (b, i, 0)),
        out_shape=jax.ShapeDtypeStruct((B, Ln, K), jnp.int32),
    )(x16, xt)

    info = plsc.get_sparse_core_info()
    nw = info.num_cores * info.num_subcores
    gidx = (eidx + (jnp.arange(B, dtype=jnp.int32) * Ln)[:, None, None])
    gidx3 = gidx.reshape(nw, (B * Ln * K) // (nw * 128), 128)
    table = x16.reshape(B * Ln, 16)
    nb_flat = _sc_gather(table, gidx3, B)

    RB = 3840
    RBL = RB // K
    p1, p2, smat, cmat = _static_mats()
    nrbf = NUM_RBF * _NP
    e_flat = pl.pallas_call(
        _edge_body,
        grid=(B, (Ln * K) // RB),
        in_specs=[
            pl.BlockSpec((1, RBL, 16), lambda b, i: (b, i, 0)),
            pl.BlockSpec((1, RB, 16), lambda b, i: (b, i, 0)),
            pl.BlockSpec((1, RBL, K), lambda b, i: (b, i, 0)),
            pl.BlockSpec(p1.shape, lambda b, i: (0, 0)),
            pl.BlockSpec(p2.shape, lambda b, i: (0, 0)),
            pl.BlockSpec(smat.shape, lambda b, i: (0, 0)),
            pl.BlockSpec(cmat.shape, lambda b, i: (0, 0)),
            pl.BlockSpec((nrbf, F), lambda b, i: (0, 0)),
            pl.BlockSpec((2 * MAX_REL + 1, F), lambda b, i: (0, 0)),
            pl.BlockSpec((1, F), lambda b, i: (0, 0)),
            pl.BlockSpec((1, F), lambda b, i: (0, 0)),
            pl.BlockSpec((1, F), lambda b, i: (0, 0)),
        ],
        out_specs=pl.BlockSpec((1, RBL, K, F), lambda b, i: (b, i, 0, 0)),
        out_shape=jax.ShapeDtypeStruct((B, Ln, K, F), jnp.float32),
    )(x16, nb_flat, eidx, p1, p2, smat, cmat, edge_W, pos_W,
      ne_g[None, :], ne_b[None, :], pos_b[None, :])

    return V, e_flat, eidx


# speedup vs baseline: 1.0773x; 1.0462x over previous
import functools

import jax
import jax.numpy as jnp
import numpy as np
from jax import lax
from jax.experimental import pallas as pl
from jax.experimental.pallas import tpu as pltpu
from jax.experimental.pallas import tpu_sc as plsc

TOP_K = 30
NUM_RBF = 16
MAX_REL = 32

_PAIRS = [(1, 1),
          (0, 0), (2, 2), (4, 4), (3, 3), (1, 0), (1, 2), (1, 4), (1, 3),
          (0, 2), (0, 4), (0, 3), (3, 2), (3, 4), (4, 2), (0, 1), (2, 1),
          (4, 1), (3, 1), (2, 0), (4, 0), (3, 0), (2, 3), (4, 3), (2, 4)]
_NP = len(_PAIRS)


def _static_mats():
    p1 = np.zeros((16, 3 * _NP), np.float32)
    p2 = np.zeros((16, 3 * _NP), np.float32)
    smat = np.zeros((3 * _NP, _NP), np.float32)
    for p, (a, b) in enumerate(_PAIRS):
        for c in range(3):
            p1[3 * a + c, 3 * p + c] = 1.0
            p2[3 * b + c, 3 * p + c] = 1.0
            smat[3 * p + c, p] = 1.0
    mu16 = 2.0 + np.arange(NUM_RBF, dtype=np.float32) * (20.0 / (NUM_RBF - 1))
    is2 = (NUM_RBF / 20.0) ** 2
    cmat = np.zeros((2 * _NP + 1, NUM_RBF * _NP), np.float32)
    for p in range(_NP):
        for j in range(NUM_RBF):
            col = NUM_RBF * p + j
            cmat[p, col] = -is2
            cmat[_NP + p, col] = 2.0 * mu16[j] * is2
            cmat[2 * _NP, col] = -mu16[j] * mu16[j] * is2
    return (jnp.asarray(p1), jnp.asarray(p2), jnp.asarray(smat),
            jnp.asarray(cmat))


def _topk_body(x16_ref, xt_ref, eidx_ref):
    x16 = x16_ref[0]
    xt = xt_ref[0]
    d2 = None
    for c in range(3):
        xb = x16[:, 3 + c:4 + c]
        xa = xt[3 + c:4 + c, :]
        df = xb - xa
        t = df * df
        d2 = t if d2 is None else d2 + t
    dcur = jnp.sqrt(d2 + 1e-6)
    iota = lax.broadcasted_iota(jnp.int32, dcur.shape, 1)
    for k in range(TOP_K):
        m = jnp.min(dcur, axis=1, keepdims=True)
        cand = jnp.where(dcur == m, iota, jnp.int32(1 << 30))
        idx = jnp.min(cand, axis=1, keepdims=True)
        eidx_ref[0, :, k:k + 1] = idx
        if k < TOP_K - 1:
            dcur = jnp.where(iota == idx, jnp.float32(1e30), dcur)


def _edge_body(own_ref, nb_ref, ei_ref, p1_ref, p2_ref, s_ref,
               c_ref, ew_ref, pw_ref, g_ref, b_ref, pb_ref, e_ref):
    ownb = own_ref[0]
    nb = nb_ref[0]
    kk = nb.shape[0] // ownb.shape[0]
    own = jnp.broadcast_to(ownb[:, None, :],
                           (ownb.shape[0], kk, 16)).reshape(nb.shape)
    own_e = jnp.dot(own, p1_ref[...], preferred_element_type=jnp.float32, precision=lax.Precision.HIGHEST)
    nb_e = jnp.dot(nb, p2_ref[...], preferred_element_type=jnp.float32, precision=lax.Precision.HIGHEST)
    diff = own_e - nb_e
    d2 = jnp.dot(diff * diff, s_ref[...], preferred_element_type=jnp.float32, precision=lax.Precision.HIGHEST)
    d2p = d2 + 1e-6
    d = jnp.sqrt(d2p)
    ones = jnp.full((nb.shape[0], 1), 1.0, jnp.float32)
    g = jnp.concatenate([d2p, d, ones], axis=1)
    zneg = jnp.dot(g, c_ref[...], preferred_element_type=jnp.float32, precision=lax.Precision.HIGHEST)
    rbf = jnp.exp(zneg)
    y = jnp.dot(rbf, ew_ref[...], preferred_element_type=jnp.float32)
    m = jnp.mean(y, axis=1, keepdims=True)
    v = jnp.mean((y - m) ** 2, axis=1, keepdims=True)
    yn = (y - m) * lax.rsqrt(v + 1e-5) * g_ref[...] + b_ref[...]
    rbl = ownb.shape[0]
    rows = (pl.program_id(1) * rbl
            + lax.broadcasted_iota(jnp.int32, (rbl, kk), 0))
    dcl = jnp.clip(rows - ei_ref[0] + MAX_REL, 0, 2 * MAX_REL)
    oh = (dcl[:, :, None]
          == lax.broadcasted_iota(jnp.int32, (1, 1, 2 * MAX_REL + 1), 2)
          ).astype(jnp.float32).reshape(nb.shape[0], 2 * MAX_REL + 1)
    pos = jnp.dot(oh, pw_ref[...], preferred_element_type=jnp.float32)
    out = yn + pos + pb_ref[...]
    e_ref[0] = out.reshape(e_ref.shape[1:])


def _node_body(xt_ref, w_ref, g_ref, b_ref, v_ref):
    x = xt_ref[0]

    def col(i):
        return x[i:i + 1, :]

    zero = jnp.zeros((1, 1), jnp.float32)

    def shift_up(v):
        return jnp.concatenate([v[:, 1:], zero], axis=1)

    def shift_down(v):
        return jnp.concatenate([zero, v[:, :-1]], axis=1)

    def norm3(v):
        n = jnp.sqrt(v[0] * v[0] + v[1] * v[1] + v[2] * v[2])
        inv = 1.0 / jnp.maximum(n, 1e-12)
        return [v[0] * inv, v[1] * inv, v[2] * inv]

    def cross(u, v):
        return [u[1] * v[2] - u[2] * v[1],
                u[2] * v[0] - u[0] * v[2],
                u[0] * v[1] - u[1] * v[0]]

    def dot3(u, v):
        return u[0] * v[0] + u[1] * v[1] + u[2] * v[2]

    n_at = [col(0), col(1), col(2)]
    ca_at = [col(3), col(4), col(5)]
    c_at = [col(6), col(7), col(8)]
    avec = norm3([ca_at[c] - n_at[c] for c in range(3)])
    bvec = norm3([c_at[c] - ca_at[c] for c in range(3)])
    cvec = norm3([shift_up(n_at[c]) - c_at[c] for c in range(3)])

    def dihed(u2, u1, u0):
        n2 = norm3(cross(u2, u1))
        n1 = norm3(cross(u1, u0))
        cosd = jnp.clip(dot3(n2, n1), -1.0 + 1e-7, 1.0 - 1e-7)
        sind = jnp.sign(dot3(u2, n1)) * jnp.sqrt(1.0 - cosd * cosd)
        return cosd, sind

    c_prev = [shift_down(cvec[i]) for i in range(3)]
    a_next = [shift_up(avec[i]) for i in range(3)]
    cph, sph = dihed(c_prev, avec, bvec)
    cps, sps = dihed(avec, bvec, cvec)
    com, som = dihed(bvec, cvec, a_next)
    r = lax.broadcasted_iota(jnp.int32, cph.shape, 1)
    first = r == 0
    last = r == (cph.shape[1] - 1)
    cph = jnp.where(first, 1.0, cph)
    sph = jnp.where(first, 0.0, sph)
    cps = jnp.where(last, 1.0, cps)
    sps = jnp.where(last, 0.0, sps)
    com = jnp.where(last, 1.0, com)
    som = jnp.where(last, 0.0, som)
    feat = jnp.concatenate([cph, cps, com, sph, sps, som], axis=0)
    y = lax.dot_general(feat, w_ref[...], (((0,), (0,)), ((), ())),
                        preferred_element_type=jnp.float32,
                        precision=lax.Precision.HIGHEST)
    m = jnp.mean(y, axis=1, keepdims=True)
    v = jnp.mean((y - m) ** 2, axis=1, keepdims=True)
    v_ref[0] = (y - m) * lax.rsqrt(v + 1e-5) * g_ref[...] + b_ref[...]


def _sc_gather(table, gidx3, batch):
    info = plsc.get_sparse_core_info()
    nc, ns = info.num_cores, info.num_subcores
    nw = nc * ns
    chunks = gidx3.shape[1]
    per_w = chunks * 128
    rows = nw * per_w
    wpb = nw // batch
    mesh = plsc.VectorSubcoreMesh(core_axis_name="c", subcore_axis_name="s")

    @functools.partial(
        pl.kernel, mesh=mesh,
        compiler_params=pltpu.CompilerParams(use_tc_tiling_on_sc=False),
        out_type=jax.ShapeDtypeStruct((batch, rows // batch, 16), jnp.float32),
        scratch_types=[
            pltpu.VMEM((chunks, 128), jnp.int32),
            pltpu.VMEM((per_w, 16), jnp.float32),
            pltpu.SemaphoreType.DMA,
            pltpu.SemaphoreType.DMA,
        ])
    def gather_k(table_hbm, idx_hbm, out_hbm, idx_v, rows_v, sem0, sem1):
        wid = lax.axis_index("s") * nc + lax.axis_index("c")
        pltpu.sync_copy(idx_hbm.at[wid], idx_v)

        def body(g, carry):
            c0 = 2 * g
            c1 = 2 * g + 1
            cp0 = pltpu.make_async_copy(
                table_hbm.at[idx_v.at[c0]],
                rows_v.at[pl.ds(c0 * 128, 128)], sem0)
            cp1 = pltpu.make_async_copy(
                table_hbm.at[idx_v.at[c1]],
                rows_v.at[pl.ds(c1 * 128, 128)], sem1)
            cp0.start()
            cp1.start()
            cp0.wait()
            cp1.wait()
            return carry

        lax.fori_loop(0, chunks // 2, body, 0)
        pltpu.sync_copy(
            rows_v, out_hbm.at[wid // wpb, pl.ds((wid % wpb) * per_w, per_w)])

    return gather_k(table, gidx3)


def kernel(X, mask, L, pos_W, pos_b, edge_W, ne_g, ne_b, node_W, nn_g, nn_b):
    B, Ln = X.shape[0], X.shape[1]
    K = TOP_K
    F = edge_W.shape[1]
    x16 = jnp.concatenate(
        [X.reshape(B, Ln, 15), jnp.zeros((B, Ln, 1), jnp.float32)], axis=-1)
    xt = jnp.transpose(x16, (0, 2, 1))

    NF = node_W.shape[1]
    V = pl.pallas_call(
        _node_body,
        grid=(B,),
        in_specs=[
            pl.BlockSpec((1, 16, Ln), lambda b: (b, 0, 0)),
            pl.BlockSpec((6, NF), lambda b: (0, 0)),
            pl.BlockSpec((1, NF), lambda b: (0, 0)),
            pl.BlockSpec((1, NF), lambda b: (0, 0)),
        ],
        out_specs=pl.BlockSpec((1, Ln, NF), lambda b: (b, 0, 0)),
        out_shape=jax.ShapeDtypeStruct((B, Ln, NF), jnp.float32),
    )(xt, node_W, nn_g[None, :], nn_b[None, :])

    BL = 128
    eidx = pl.pallas_call(
        _topk_body,
        grid=(B, Ln // BL),
        in_specs=[
            pl.BlockSpec((1, BL, 16), lambda b, i: (b, i, 0)),
            pl.BlockSpec((1, 16, Ln), lambda b, i: (b, 0, 0)),
        ],
        out_specs=pl.BlockSpec((1, BL, K), lambda b, i: (b, i, 0)),
        out_shape=jax.ShapeDtypeStruct((B, Ln, K), jnp.int32),
    )(x16, xt)

    info = plsc.get_sparse_core_info()
    nw = info.num_cores * info.num_subcores
    gidx = (eidx + (jnp.arange(B, dtype=jnp.int32) * Ln)[:, None, None])
    gidx3 = gidx.reshape(nw, (B * Ln * K) // (nw * 128), 128)
    table = x16.reshape(B * Ln, 16)
    nb_flat = _sc_gather(table, gidx3, B)

    RB = 3840
    RBL = RB // K
    p1, p2, smat, cmat = _static_mats()
    nrbf = NUM_RBF * _NP
    e_flat = pl.pallas_call(
        _edge_body,
        grid=(B, (Ln * K) // RB),
        in_specs=[
            pl.BlockSpec((1, RBL, 16), lambda b, i: (b, i, 0)),
            pl.BlockSpec((1, RB, 16), lambda b, i: (b, i, 0)),
            pl.BlockSpec((1, RBL, K), lambda b, i: (b, i, 0)),
            pl.BlockSpec(p1.shape, lambda b, i: (0, 0)),
            pl.BlockSpec(p2.shape, lambda b, i: (0, 0)),
            pl.BlockSpec(smat.shape, lambda b, i: (0, 0)),
            pl.BlockSpec(cmat.shape, lambda b, i: (0, 0)),
            pl.BlockSpec((nrbf, F), lambda b, i: (0, 0)),
            pl.BlockSpec((2 * MAX_REL + 1, F), lambda b, i: (0, 0)),
            pl.BlockSpec((1, F), lambda b, i: (0, 0)),
            pl.BlockSpec((1, F), lambda b, i: (0, 0)),
            pl.BlockSpec((1, F), lambda b, i: (0, 0)),
        ],
        out_specs=pl.BlockSpec((1, RBL, K, F), lambda b, i: (b, i, 0, 0)),
        out_shape=jax.ShapeDtypeStruct((B, Ln, K, F), jnp.float32),
    )(x16, nb_flat, eidx, p1, p2, smat, cmat, edge_W, pos_W,
      ne_g[None, :], ne_b[None, :], pos_b[None, :])

    return V, e_flat, eidx
